# Initial kernel scaffold; baseline (speedup 1.0000x reference)
import jax
import jax.numpy as jnp
from jax.experimental import pallas as pl


def kernel(x, edge_index, pruned_values, W0, W1, Wm, eps0, eps1):
    raise NotImplementedError("write your pallas kernel here")


# speedup vs baseline: 3.8584x; 3.8584x over previous
import functools

import jax
import jax.numpy as jnp
from jax import lax
from jax.experimental import pallas as pl
from jax.experimental.pallas import tpu as pltpu
from jax.experimental.pallas import tpu_sc as plsc

N = 10000
D = 128
E = 320000
NC = 2
NS = 16
EPC = E // NC
EPT = EPC // NS
CH = 80
NCHUNK = EPT // CH
RPT = 632
NPAD = NS * RPT
LANES = 16

_sc_mesh = plsc.VectorSubcoreMesh(
    core_axis_name="c", subcore_axis_name="s", num_cores=NC, num_subcores=NS)


@functools.partial(
    pl.kernel,
    out_type=jax.ShapeDtypeStruct((NC, NPAD, D), jnp.float32),
    mesh=_sc_mesh,
    scratch_types=[
        pltpu.VMEM((CH,), jnp.int32),
        pltpu.VMEM((CH,), jnp.int32),
        pltpu.VMEM((CH,), jnp.float32),
        pltpu.VMEM((CH, D), jnp.float32),
        pltpu.VMEM_SHARED((NPAD, D), jnp.float32),
        pltpu.SemaphoreType.DMA,
    ],
)
def _sc_aggregate(feat_hbm, col_hbm, row_hbm, w_hbm, zeros_hbm, out_hbm,
                  colv, rowv, wv, rowsv, acc, sem):
  c = lax.axis_index("c")
  s = lax.axis_index("s")
  pltpu.sync_copy(zeros_hbm, acc.at[pl.ds(s * RPT, RPT)])
  plsc.subcore_barrier()

  tile_base = c * EPC + s * EPT

  def chunk_body(k, carry):
    base = tile_base + k * CH
    pltpu.sync_copy(col_hbm.at[pl.ds(base, CH)], colv)
    pltpu.sync_copy(row_hbm.at[pl.ds(base, CH)], rowv)
    pltpu.sync_copy(w_hbm.at[pl.ds(base, CH)], wv)
    pltpu.async_copy(feat_hbm.at[colv], rowsv, sem).wait()

    def scale_group(g, carry2):
      wvec = wv[pl.ds(g * LANES, LANES)]
      for i in range(LANES):
        we = wvec[i]
        e = g * LANES + i
        for j in range(D // LANES):
          sl = pl.ds(j * LANES, LANES)
          rowsv[e, sl] = rowsv[e, sl] * we
      return carry2

    lax.fori_loop(0, CH // LANES, scale_group, 0)
    pltpu.sync_copy(rowsv, acc.at[rowv], add=True)
    return carry

  lax.fori_loop(0, NCHUNK, chunk_body, 0)
  plsc.subcore_barrier()
  pltpu.sync_copy(acc.at[pl.ds(s * RPT, RPT)],
                  out_hbm.at[c, pl.ds(s * RPT, RPT)])


_ROWS_BLK = 1000


def _matmul_t(a, w_ref):
  return lax.dot_general(a, w_ref[...], (((1,), (1,)), ((), ())),
                         preferred_element_type=jnp.float32)


def _layer_body(p_ref, f_ref, w_ref, s_ref, o_ref):
  a = p_ref[0] + p_ref[1] + s_ref[0, 0] * f_ref[...]
  o_ref[...] = jnp.maximum(_matmul_t(a, w_ref), 0.0)


def _gin_layer_tc(partials, feat, W, eps):
  scale = (1.0 + eps).reshape(1, 1)
  return pl.pallas_call(
      _layer_body,
      grid=(N // _ROWS_BLK,),
      in_specs=[
          pl.BlockSpec((NC, _ROWS_BLK, D), lambda k: (0, k, 0)),
          pl.BlockSpec((_ROWS_BLK, D), lambda k: (k, 0)),
          pl.BlockSpec((D, D), lambda k: (0, 0)),
          pl.BlockSpec(memory_space=pltpu.SMEM),
      ],
      out_specs=pl.BlockSpec((_ROWS_BLK, D), lambda k: (k, 0)),
      out_shape=jax.ShapeDtypeStruct((N, D), jnp.float32),
  )(partials, feat, W, scale)


def _out_body(p_ref, h_ref, w1_ref, wm_ref, s_ref, o_ref):
  h0 = h_ref[...]
  a = p_ref[0] + p_ref[1] + s_ref[0, 0] * h0
  h1 = jnp.maximum(_matmul_t(a, w1_ref), 0.0)
  o_ref[...] = (_matmul_t(h0, wm_ref) + h1) * 0.5


def _gin_out_tc(partials, h0, W1, Wm, eps1):
  scale = (1.0 + eps1).reshape(1, 1)
  return pl.pallas_call(
      _out_body,
      grid=(N // _ROWS_BLK,),
      in_specs=[
          pl.BlockSpec((NC, _ROWS_BLK, D), lambda k: (0, k, 0)),
          pl.BlockSpec((_ROWS_BLK, D), lambda k: (k, 0)),
          pl.BlockSpec((D, D), lambda k: (0, 0)),
          pl.BlockSpec((D, D), lambda k: (0, 0)),
          pl.BlockSpec(memory_space=pltpu.SMEM),
      ],
      out_specs=pl.BlockSpec((_ROWS_BLK, D), lambda k: (k, 0)),
      out_shape=jax.ShapeDtypeStruct((N, D), jnp.float32),
  )(partials, h0, W1, Wm, scale)


def kernel(x, edge_index, pruned_values, W0, W1, Wm, eps0, eps1):
  row = edge_index[0]
  col = edge_index[1]
  w = pruned_values
  zeros = jnp.zeros((RPT, D), jnp.float32)

  p0 = _sc_aggregate(x, col, row, w, zeros)
  h0 = _gin_layer_tc(p0, x, W0, eps0)
  p1 = _sc_aggregate(h0, col, row, w, zeros)
  return _gin_out_tc(p1, h0, W1, Wm, eps1)


# speedup vs baseline: 9.9486x; 2.5785x over previous
import functools

import jax
import jax.numpy as jnp
from jax import lax
from jax.experimental import pallas as pl
from jax.experimental.pallas import tpu as pltpu
from jax.experimental.pallas import tpu_sc as plsc

N = 10000
D = 128
E = 320000
NC = 2
NS = 16
EPC = E // NC
EPT = EPC // NS
CH = 80
NCHUNK = EPT // CH
RPT = 632
NPAD = NS * RPT
LANES = 16

_sc_mesh = plsc.VectorSubcoreMesh(
    core_axis_name="c", subcore_axis_name="s", num_cores=NC, num_subcores=NS)


@functools.partial(
    pl.kernel,
    out_type=jax.ShapeDtypeStruct((NC, NPAD, D), jnp.float32),
    mesh=_sc_mesh,
    scratch_types=[
        pltpu.VMEM((EPT,), jnp.int32),
        pltpu.VMEM((CH,), jnp.int32),
        pltpu.VMEM((CH,), jnp.int32),
        pltpu.VMEM((CH,), jnp.float32),
        pltpu.VMEM((CH,), jnp.float32),
        pltpu.VMEM((CH, D), jnp.float32),
        pltpu.VMEM((CH, D), jnp.float32),
        pltpu.VMEM_SHARED((NPAD, D), jnp.float32),
        pltpu.SemaphoreType.DMA,
        pltpu.SemaphoreType.DMA,
    ],
)
def _sc_aggregate(feat_hbm, col_hbm, row_hbm, w_hbm, zeros_hbm, out_hbm,
                  cols, rowvA, rowvB, wvA, wvB, bufA, bufB, acc, semA, semB):
  c = lax.axis_index("c")
  s = lax.axis_index("s")
  pltpu.sync_copy(zeros_hbm, acc.at[pl.ds(s * RPT, RPT)])

  tile_base = c * EPC + s * EPT
  pltpu.sync_copy(col_hbm.at[pl.ds(tile_base, EPT)], cols)
  plsc.subcore_barrier()

  def issue(k, buf, rowv, wv, sem):
    base = tile_base + k * CH
    pltpu.async_copy(row_hbm.at[pl.ds(base, CH)], rowv, sem)
    pltpu.async_copy(w_hbm.at[pl.ds(base, CH)], wv, sem)
    pltpu.async_copy(feat_hbm.at[cols.at[pl.ds(k * CH, CH)]], buf, sem)

  def drain(buf, rowv, wv, sem):
    pltpu.make_async_copy(row_hbm.at[pl.ds(0, CH)], rowv, sem).wait()
    pltpu.make_async_copy(w_hbm.at[pl.ds(0, CH)], wv, sem).wait()
    pltpu.make_async_copy(feat_hbm.at[pl.ds(0, CH)], buf, sem).wait()

  def process(buf, rowv, wv):
    def scale_group(g, carry2):
      wvec = wv[pl.ds(g * LANES, LANES)]
      for i in range(LANES):
        we = wvec[i]
        e = g * LANES + i
        for j in range(D // LANES):
          sl = pl.ds(j * LANES, LANES)
          buf[e, sl] = buf[e, sl] * we
      return carry2

    lax.fori_loop(0, CH // LANES, scale_group, 0)
    pltpu.sync_copy(buf, acc.at[rowv], add=True)

  issue(0, bufA, rowvA, wvA, semA)

  def pipe_body(i, carry):
    k0 = i * 2
    issue(k0 + 1, bufB, rowvB, wvB, semB)
    drain(bufA, rowvA, wvA, semA)
    process(bufA, rowvA, wvA)
    issue(k0 + 2, bufA, rowvA, wvA, semA)
    drain(bufB, rowvB, wvB, semB)
    process(bufB, rowvB, wvB)
    return carry

  lax.fori_loop(0, (NCHUNK - 1) // 2, pipe_body, 0)
  drain(bufA, rowvA, wvA, semA)
  process(bufA, rowvA, wvA)

  plsc.subcore_barrier()
  pltpu.sync_copy(acc.at[pl.ds(s * RPT, RPT)],
                  out_hbm.at[c, pl.ds(s * RPT, RPT)])


_ROWS_BLK = 1000


def _matmul_t(a, w_ref):
  return lax.dot_general(a, w_ref[...], (((1,), (1,)), ((), ())),
                         preferred_element_type=jnp.float32)


def _layer_body(p_ref, f_ref, w_ref, s_ref, o_ref):
  a = p_ref[0] + p_ref[1] + s_ref[0, 0] * f_ref[...]
  o_ref[...] = jnp.maximum(_matmul_t(a, w_ref), 0.0)


def _gin_layer_tc(partials, feat, W, eps):
  scale = (1.0 + eps).reshape(1, 1)
  return pl.pallas_call(
      _layer_body,
      grid=(N // _ROWS_BLK,),
      in_specs=[
          pl.BlockSpec((NC, _ROWS_BLK, D), lambda k: (0, k, 0)),
          pl.BlockSpec((_ROWS_BLK, D), lambda k: (k, 0)),
          pl.BlockSpec((D, D), lambda k: (0, 0)),
          pl.BlockSpec(memory_space=pltpu.SMEM),
      ],
      out_specs=pl.BlockSpec((_ROWS_BLK, D), lambda k: (k, 0)),
      out_shape=jax.ShapeDtypeStruct((N, D), jnp.float32),
  )(partials, feat, W, scale)


def _out_body(p_ref, h_ref, w1_ref, wm_ref, s_ref, o_ref):
  h0 = h_ref[...]
  a = p_ref[0] + p_ref[1] + s_ref[0, 0] * h0
  h1 = jnp.maximum(_matmul_t(a, w1_ref), 0.0)
  o_ref[...] = (_matmul_t(h0, wm_ref) + h1) * 0.5


def _gin_out_tc(partials, h0, W1, Wm, eps1):
  scale = (1.0 + eps1).reshape(1, 1)
  return pl.pallas_call(
      _out_body,
      grid=(N // _ROWS_BLK,),
      in_specs=[
          pl.BlockSpec((NC, _ROWS_BLK, D), lambda k: (0, k, 0)),
          pl.BlockSpec((_ROWS_BLK, D), lambda k: (k, 0)),
          pl.BlockSpec((D, D), lambda k: (0, 0)),
          pl.BlockSpec((D, D), lambda k: (0, 0)),
          pl.BlockSpec(memory_space=pltpu.SMEM),
      ],
      out_specs=pl.BlockSpec((_ROWS_BLK, D), lambda k: (k, 0)),
      out_shape=jax.ShapeDtypeStruct((N, D), jnp.float32),
  )(partials, h0, W1, Wm, scale)


def kernel(x, edge_index, pruned_values, W0, W1, Wm, eps0, eps1):
  row = edge_index[0]
  col = edge_index[1]
  w = pruned_values
  zeros = jnp.zeros((RPT, D), jnp.float32)

  p0 = _sc_aggregate(x, col, row, w, zeros)
  h0 = _gin_layer_tc(p0, x, W0, eps0)
  p1 = _sc_aggregate(h0, col, row, w, zeros)
  return _gin_out_tc(p1, h0, W1, Wm, eps1)


# speedup vs baseline: 11.0586x; 1.1116x over previous
import functools

import jax
import jax.numpy as jnp
from jax import lax
from jax.experimental import pallas as pl
from jax.experimental.pallas import tpu as pltpu
from jax.experimental.pallas import tpu_sc as plsc

N = 10000
D = 128
E = 320000
NC = 2
NS = 16
EPC = E // NC
EPT = EPC // NS
CH = 80
NCHUNK = EPT // CH
RPT = 632
NPAD = NS * RPT
LANES = 16

_sc_mesh = plsc.VectorSubcoreMesh(
    core_axis_name="c", subcore_axis_name="s", num_cores=NC, num_subcores=NS)


@functools.partial(
    pl.kernel,
    out_type=jax.ShapeDtypeStruct((NC, NPAD, D), jnp.float32),
    mesh=_sc_mesh,
    scratch_types=[
        pltpu.VMEM((EPT,), jnp.int32),
        [pltpu.VMEM((CH,), jnp.int32) for _ in range(3)],
        [pltpu.VMEM((CH,), jnp.float32) for _ in range(3)],
        [pltpu.VMEM((CH, D), jnp.float32) for _ in range(3)],
        pltpu.VMEM_SHARED((NPAD, D), jnp.float32),
        [pltpu.SemaphoreType.DMA for _ in range(3)],
        [pltpu.SemaphoreType.DMA for _ in range(3)],
    ],
)
def _sc_aggregate(feat_hbm, col_hbm, row_hbm, w_hbm, zeros_hbm, out_hbm,
                  cols, rowv, wv, buf, acc, lsem, ssem):
  c = lax.axis_index("c")
  s = lax.axis_index("s")
  pltpu.sync_copy(zeros_hbm, acc.at[pl.ds(s * RPT, RPT)])

  tile_base = c * EPC + s * EPT
  pltpu.sync_copy(col_hbm.at[pl.ds(tile_base, EPT)], cols)
  plsc.subcore_barrier()

  def issue_loads(k, sl):
    base = tile_base + k * CH
    pltpu.async_copy(row_hbm.at[pl.ds(base, CH)], rowv[sl], lsem[sl])
    pltpu.async_copy(w_hbm.at[pl.ds(base, CH)], wv[sl], lsem[sl])
    pltpu.async_copy(feat_hbm.at[cols.at[pl.ds(k * CH, CH)]], buf[sl],
                     lsem[sl])

  def drain_loads(sl):
    pltpu.make_async_copy(row_hbm.at[pl.ds(0, CH)], rowv[sl], lsem[sl]).wait()
    pltpu.make_async_copy(w_hbm.at[pl.ds(0, CH)], wv[sl], lsem[sl]).wait()
    pltpu.make_async_copy(feat_hbm.at[pl.ds(0, CH)], buf[sl], lsem[sl]).wait()

  def scale(sl):
    def scale_group(g, carry2):
      wvec = wv[sl][pl.ds(g * LANES, LANES)]
      for i in range(LANES):
        we = wvec[i]
        e = g * LANES + i
        for j in range(D // LANES):
          slc = pl.ds(j * LANES, LANES)

          buf[sl][e, slc] = buf[sl][e, slc] * we
      return carry2

    lax.fori_loop(0, CH // LANES, scale_group, 0)

  def issue_scatter(sl):
    pltpu.async_copy(buf[sl], acc.at[rowv[sl]], ssem[sl], add=True)

  def drain_scatter(sl):
    pltpu.make_async_copy(feat_hbm.at[pl.ds(0, CH)], buf[sl],
                          ssem[sl]).wait()

  issue_loads(0, 0)
  issue_loads(1, 1)
  drain_loads(0)
  scale(0)
  issue_scatter(0)
  issue_loads(2, 2)
  drain_loads(1)
  scale(1)
  issue_scatter(1)
  drain_scatter(0)
  issue_loads(3, 0)

  def steady(k, sl):
    drain_loads(sl)
    scale(sl)
    issue_scatter(sl)
    drain_scatter((sl + 2) % 3)
    issue_loads(k + 2, (sl + 2) % 3)

  def pipe_body(i, carry):
    k = 3 * i + 2
    steady(k, 2)
    steady(k + 1, 0)
    steady(k + 2, 1)
    return carry

  lax.fori_loop(0, (NCHUNK - 5) // 3, pipe_body, 0)

  drain_loads(2); scale(2); issue_scatter(2); drain_scatter(1)
  issue_loads(NCHUNK - 1, 1)
  drain_loads(0); scale(0); issue_scatter(0); drain_scatter(2)
  drain_loads(1); scale(1); issue_scatter(1); drain_scatter(0)
  drain_scatter(1)

  plsc.subcore_barrier()
  pltpu.sync_copy(acc.at[pl.ds(s * RPT, RPT)],
                  out_hbm.at[c, pl.ds(s * RPT, RPT)])


_ROWS_BLK = 1000


def _matmul_t(a, w_ref):
  return lax.dot_general(a, w_ref[...], (((1,), (1,)), ((), ())),
                         preferred_element_type=jnp.float32)


def _layer_body(p_ref, f_ref, w_ref, s_ref, o_ref):
  a = p_ref[0] + p_ref[1] + s_ref[0, 0] * f_ref[...]
  o_ref[...] = jnp.maximum(_matmul_t(a, w_ref), 0.0)


def _gin_layer_tc(partials, feat, W, eps):
  scale = (1.0 + eps).reshape(1, 1)
  return pl.pallas_call(
      _layer_body,
      grid=(N // _ROWS_BLK,),
      in_specs=[
          pl.BlockSpec((NC, _ROWS_BLK, D), lambda k: (0, k, 0)),
          pl.BlockSpec((_ROWS_BLK, D), lambda k: (k, 0)),
          pl.BlockSpec((D, D), lambda k: (0, 0)),
          pl.BlockSpec(memory_space=pltpu.SMEM),
      ],
      out_specs=pl.BlockSpec((_ROWS_BLK, D), lambda k: (k, 0)),
      out_shape=jax.ShapeDtypeStruct((N, D), jnp.float32),
  )(partials, feat, W, scale)


def _out_body(p_ref, h_ref, w1_ref, wm_ref, s_ref, o_ref):
  h0 = h_ref[...]
  a = p_ref[0] + p_ref[1] + s_ref[0, 0] * h0
  h1 = jnp.maximum(_matmul_t(a, w1_ref), 0.0)
  o_ref[...] = (_matmul_t(h0, wm_ref) + h1) * 0.5


def _gin_out_tc(partials, h0, W1, Wm, eps1):
  scale = (1.0 + eps1).reshape(1, 1)
  return pl.pallas_call(
      _out_body,
      grid=(N // _ROWS_BLK,),
      in_specs=[
          pl.BlockSpec((NC, _ROWS_BLK, D), lambda k: (0, k, 0)),
          pl.BlockSpec((_ROWS_BLK, D), lambda k: (k, 0)),
          pl.BlockSpec((D, D), lambda k: (0, 0)),
          pl.BlockSpec((D, D), lambda k: (0, 0)),
          pl.BlockSpec(memory_space=pltpu.SMEM),
      ],
      out_specs=pl.BlockSpec((_ROWS_BLK, D), lambda k: (k, 0)),
      out_shape=jax.ShapeDtypeStruct((N, D), jnp.float32),
  )(partials, h0, W1, Wm, scale)


def kernel(x, edge_index, pruned_values, W0, W1, Wm, eps0, eps1):
  row = edge_index[0]
  col = edge_index[1]
  w = pruned_values
  zeros = jnp.zeros((RPT, D), jnp.float32)

  p0 = _sc_aggregate(x, col, row, w, zeros)
  h0 = _gin_layer_tc(p0, x, W0, eps0)
  p1 = _sc_aggregate(h0, col, row, w, zeros)
  return _gin_out_tc(p1, h0, W1, Wm, eps1)
